# baseline (speedup 1.0000x reference)
import jax
import jax.numpy as jnp
from jax.experimental import pallas as pl
from jax.experimental.pallas import tpu as pltpu

_N = 20000
_BR = 2560
_BLOCKS = [(0, 1280), (1280, 1280)]
_o = 2560
while _N - _o > _BR:
    _BLOCKS.append((_o, _BR))
    _o += _BR
_BLOCKS.append((_o, _N - _o))
_NB = len(_BLOCKS)
_R_LAST = _N - _o

_DN = (((1,), (1,)), ((), ()))


def _kernel(x_hbm, wc_ref, bc_ref, wb_ref, bb_ref, st_hbm, dt_hbm,
            xb0, xb1, sb0, sb1, db0, db1, sbl, dbl, in_sem, os_sem, od_sem):
    xbufs = (xb0, xb1)
    sbufs = (sb0, sb1)
    dbufs = (db0, db1)

    def in_copies(i):
        o, r = _BLOCKS[i]
        h = r // 2
        return (
            pltpu.make_async_copy(
                x_hbm.at[pl.ds(o, h), :],
                xbufs[i % 2].at[pl.ds(0, h), :],
                in_sem.at[i % 2, 0],
            ),
            pltpu.make_async_copy(
                x_hbm.at[pl.ds(o + h, r - h), :],
                xbufs[i % 2].at[pl.ds(h, r - h), :],
                in_sem.at[i % 2, 1],
            ),
        )

    def s_copy(i):
        o, r = _BLOCKS[i]
        if i < _NB - 1:
            return pltpu.make_async_copy(
                sbufs[i % 2].at[:, pl.ds(0, r)],
                st_hbm.at[:, pl.ds(o, r)],
                os_sem.at[i % 2])
        return pltpu.make_async_copy(
            sbl, st_hbm.at[:, pl.ds(o, r)], os_sem.at[i % 2])

    def d_copy(i):
        o, r = _BLOCKS[i]
        if i < _NB - 1:
            return pltpu.make_async_copy(
                dbufs[i % 2].at[:, pl.ds(0, r)],
                dt_hbm.at[:, pl.ds(o, r)],
                od_sem.at[i % 2])
        return pltpu.make_async_copy(
            dbl, dt_hbm.at[:, pl.ds(o, r)], od_sem.at[i % 2])

    bc = bc_ref[...].T
    bb = bb_ref[...].T
    wc = wc_ref[...]
    wb = wb_ref[...]

    for c in in_copies(0):
        c.start()
    for i in range(_NB):
        if i + 1 < _NB:
            for c in in_copies(i + 1):
                c.start()
        for c in in_copies(i):
            c.wait()
        if i >= 2:
            s_copy(i - 2).wait()
            d_copy(i - 2).wait()
        s = i % 2
        r = _BLOCKS[i][1]
        x = xbufs[s][pl.ds(0, r), :]
        if i < _NB - 1:
            sbufs[s][:, pl.ds(0, r)] = jax.lax.dot_general(
                wc, x, _DN, preferred_element_type=jnp.float32) + bc
            dbufs[s][:, pl.ds(0, r)] = jax.lax.dot_general(
                wb, x, _DN, preferred_element_type=jnp.float32) + bb
        else:
            sbl[...] = jax.lax.dot_general(
                wc, x, _DN, preferred_element_type=jnp.float32) + bc
            dbl[...] = jax.lax.dot_general(
                wb, x, _DN, preferred_element_type=jnp.float32) + bb
        s_copy(i).start()
        d_copy(i).start()
    for i in (_NB - 2, _NB - 1):
        s_copy(i).wait()
        d_copy(i).wait()


@jax.jit
def kernel(x, W_cls, b_cls, W_box, b_box):
    if x.ndim > 2:
        x = x.reshape(x.shape[0], -1)
    n, in_dim = x.shape
    n_cls = W_cls.shape[0]
    n_box = W_box.shape[0]

    bc = b_cls.reshape(1, n_cls)
    bb = b_box.reshape(1, n_box)

    scores_t, deltas_t = pl.pallas_call(
        _kernel,
        in_specs=[
            pl.BlockSpec(memory_space=pltpu.HBM),
            pl.BlockSpec(memory_space=pltpu.VMEM),
            pl.BlockSpec(memory_space=pltpu.VMEM),
            pl.BlockSpec(memory_space=pltpu.VMEM),
            pl.BlockSpec(memory_space=pltpu.VMEM),
        ],
        out_specs=[
            pl.BlockSpec(memory_space=pltpu.HBM),
            pl.BlockSpec(memory_space=pltpu.HBM),
        ],
        out_shape=[
            jax.ShapeDtypeStruct((n_cls, n), jnp.float32),
            jax.ShapeDtypeStruct((n_box, n), jnp.float32),
        ],
        scratch_shapes=[
            pltpu.VMEM((_BR, in_dim), jnp.float32),
            pltpu.VMEM((_BR, in_dim), jnp.float32),
            pltpu.VMEM((n_cls, _BR), jnp.float32),
            pltpu.VMEM((n_cls, _BR), jnp.float32),
            pltpu.VMEM((n_box, _BR), jnp.float32),
            pltpu.VMEM((n_box, _BR), jnp.float32),
            pltpu.VMEM((n_cls, _R_LAST), jnp.float32),
            pltpu.VMEM((n_box, _R_LAST), jnp.float32),
            pltpu.SemaphoreType.DMA((2, 2)),
            pltpu.SemaphoreType.DMA((2,)),
            pltpu.SemaphoreType.DMA((2,)),
        ],
    )(x, W_cls, bc, W_box, bb)
    return (scores_t.T, deltas_t.T)


# speedup vs baseline: 1.0168x; 1.0168x over previous
import jax
import jax.numpy as jnp
from jax.experimental import pallas as pl
from jax.experimental.pallas import tpu as pltpu

_N = 20000
_BR = 2560
_BLOCKS = [(0, 1280)]
_o = 1280
while _N - _o > _BR:
    _BLOCKS.append((_o, _BR))
    _o += _BR
_BLOCKS.append((_o, _N - _o))
_NB = len(_BLOCKS)
_R_LAST = _N - _o

_DN = (((1,), (1,)), ((), ()))


def _kernel(x_hbm, wc_ref, bc_ref, wb_ref, bb_ref, st_hbm, dt_hbm,
            xb0, xb1, sb0, sb1, db0, db1, sbl, dbl, in_sem, os_sem, od_sem):
    xbufs = (xb0, xb1)
    sbufs = (sb0, sb1)
    dbufs = (db0, db1)

    def in_copies(i):
        o, r = _BLOCKS[i]
        h = r // 2
        return (
            pltpu.make_async_copy(
                x_hbm.at[pl.ds(o, h), :],
                xbufs[i % 2].at[pl.ds(0, h), :],
                in_sem.at[i % 2, 0],
            ),
            pltpu.make_async_copy(
                x_hbm.at[pl.ds(o + h, r - h), :],
                xbufs[i % 2].at[pl.ds(h, r - h), :],
                in_sem.at[i % 2, 1],
            ),
        )

    def s_copy(i):
        o, r = _BLOCKS[i]
        if i < _NB - 1:
            return pltpu.make_async_copy(
                sbufs[i % 2].at[:, pl.ds(0, r)],
                st_hbm.at[:, pl.ds(o, r)],
                os_sem.at[i % 2])
        return pltpu.make_async_copy(
            sbl, st_hbm.at[:, pl.ds(o, r)], os_sem.at[i % 2])

    def d_copy(i):
        o, r = _BLOCKS[i]
        if i < _NB - 1:
            return pltpu.make_async_copy(
                dbufs[i % 2].at[:, pl.ds(0, r)],
                dt_hbm.at[:, pl.ds(o, r)],
                od_sem.at[i % 2])
        return pltpu.make_async_copy(
            dbl, dt_hbm.at[:, pl.ds(o, r)], od_sem.at[i % 2])

    bc = bc_ref[...].T
    bb = bb_ref[...].T
    wc = wc_ref[...]
    wb = wb_ref[...]

    for c in in_copies(0):
        c.start()
    for i in range(_NB):
        if i + 1 < _NB:
            for c in in_copies(i + 1):
                c.start()
        for c in in_copies(i):
            c.wait()
        if i >= 2:
            s_copy(i - 2).wait()
            d_copy(i - 2).wait()
        s = i % 2
        r = _BLOCKS[i][1]
        x = xbufs[s][pl.ds(0, r), :]
        if i < _NB - 1:
            sbufs[s][:, pl.ds(0, r)] = jax.lax.dot_general(
                wc, x, _DN, preferred_element_type=jnp.float32) + bc
            dbufs[s][:, pl.ds(0, r)] = jax.lax.dot_general(
                wb, x, _DN, preferred_element_type=jnp.float32) + bb
        else:
            sbl[...] = jax.lax.dot_general(
                wc, x, _DN, preferred_element_type=jnp.float32) + bc
            dbl[...] = jax.lax.dot_general(
                wb, x, _DN, preferred_element_type=jnp.float32) + bb
        s_copy(i).start()
        d_copy(i).start()
    for i in (_NB - 2, _NB - 1):
        s_copy(i).wait()
        d_copy(i).wait()


@jax.jit
def kernel(x, W_cls, b_cls, W_box, b_box):
    if x.ndim > 2:
        x = x.reshape(x.shape[0], -1)
    n, in_dim = x.shape
    n_cls = W_cls.shape[0]
    n_box = W_box.shape[0]

    bc = b_cls.reshape(1, n_cls)
    bb = b_box.reshape(1, n_box)

    scores_t, deltas_t = pl.pallas_call(
        _kernel,
        in_specs=[
            pl.BlockSpec(memory_space=pltpu.HBM),
            pl.BlockSpec(memory_space=pltpu.VMEM),
            pl.BlockSpec(memory_space=pltpu.VMEM),
            pl.BlockSpec(memory_space=pltpu.VMEM),
            pl.BlockSpec(memory_space=pltpu.VMEM),
        ],
        out_specs=[
            pl.BlockSpec(memory_space=pltpu.HBM),
            pl.BlockSpec(memory_space=pltpu.HBM),
        ],
        out_shape=[
            jax.ShapeDtypeStruct((n_cls, n), jnp.float32),
            jax.ShapeDtypeStruct((n_box, n), jnp.float32),
        ],
        scratch_shapes=[
            pltpu.VMEM((_BR, in_dim), jnp.float32),
            pltpu.VMEM((_BR, in_dim), jnp.float32),
            pltpu.VMEM((n_cls, _BR), jnp.float32),
            pltpu.VMEM((n_cls, _BR), jnp.float32),
            pltpu.VMEM((n_box, _BR), jnp.float32),
            pltpu.VMEM((n_box, _BR), jnp.float32),
            pltpu.VMEM((n_cls, _R_LAST), jnp.float32),
            pltpu.VMEM((n_box, _R_LAST), jnp.float32),
            pltpu.SemaphoreType.DMA((2, 2)),
            pltpu.SemaphoreType.DMA((2,)),
            pltpu.SemaphoreType.DMA((2,)),
        ],
    )(x, W_cls, bc, W_box, bb)
    return (scores_t.T, deltas_t.T)
